# baseline (speedup 1.0000x reference)
import functools
import jax
import jax.numpy as jnp
from jax import lax
from jax.experimental import pallas as pl
from jax.experimental.pallas import tpu as pltpu
from jax.experimental.pallas import tpu_sc as plsc

_A = 16
_L = 16
_NW = 32
_P = 512
_G = _P // _L
_ISLAB = _P * _A
_DSLAB = _P * 2 * _A


def _body(rast, deriv, attrs, inds, out_img, out_der,
          ra, rb, dva, dvb,
          tri_b, ind_b, i0_b, i1_b, i2_b,
          a0_b, a1_b, a2_b,
          scal_a, scal_b,
          img_a, img_c, dout_a, dout_c,
          sem_in, sem_ind, sem_attr, sem_out):
    n_pix = rast.shape[0] // 4
    n_tri = inds.shape[0]
    n_vtx = attrs.shape[0]
    wid = lax.axis_index("s") * 2 + lax.axis_index("c")
    n_lines = (n_pix // _NW) // _P
    n_super = n_lines // 2
    base = wid * n_lines
    iota = lax.iota(jnp.int32, _L)

    def issue_in(line, rbuf, dbuf):
        pltpu.async_copy(rast.at[pl.ds(line * 4 * _P, 4 * _P)], rbuf, sem_in)
        pltpu.async_copy(deriv.at[pl.ds(line * 4 * _P, 4 * _P)], dbuf, sem_in)

    def drain_in(rbuf, dbuf):
        pltpu.make_async_copy(rast.at[pl.ds(0, 4 * _P)], rbuf, sem_in).wait()
        pltpu.make_async_copy(deriv.at[pl.ds(0, 4 * _P)], dbuf, sem_in).wait()

    def drain_attr():
        for ab in (a0_b, a1_b, a2_b):
            pltpu.make_async_copy(attrs.at[pl.ds(0, _P)], ab, sem_attr).wait()

    def drain_out(ibuf, obuf):
        pltpu.make_async_copy(ibuf, out_img.at[pl.ds(0, _ISLAB)], sem_out).wait()
        pltpu.make_async_copy(obuf, out_der.at[pl.ds(0, _DSLAB)], sem_out).wait()

    def front(rbuf, dbuf, scal):
        drain_in(rbuf, dbuf)

        @plsc.parallel_loop(0, _G, unroll=4)
        def fields(g):
            off = (g // 8) * 512 + (g % 8) * _L
            trif = rbuf[pl.ds(off + 3 * 128, _L)]
            tri_i = trif.astype(jnp.int32)
            validf = jnp.where(tri_i > 0, 1.0, 0.0).astype(jnp.float32)
            tric = jnp.clip(tri_i - 1, 0, n_tri - 1)
            tri_b[pl.ds(g * _L, _L)] = tric
            u = rbuf[pl.ds(off, _L)]
            v = rbuf[pl.ds(off + 128, _L)]
            w = 1.0 - u - v
            scal[0, pl.ds(g * _L, _L)] = u * validf
            scal[1, pl.ds(g * _L, _L)] = v * validf
            scal[2, pl.ds(g * _L, _L)] = w * validf
            for k in range(4):
                dk = dbuf[pl.ds(off + k * 128, _L)]
                scal[3 + k, pl.ds(g * _L, _L)] = dk * validf

        return [pltpu.async_copy(
            inds.at[tri_b.at[pl.ds(j * 128, 128)]],
            ind_b.at[pl.ds(j * 128, 128)], sem_ind) for j in range(4)]

    def mid(ind_cps):
        for cp in ind_cps:
            cp.wait()

        @plsc.parallel_loop(0, _G, unroll=4)
        def split(g):
            rows = g * _L + iota
            for k, ib in ((0, i0_b), (1, i1_b), (2, i2_b)):
                col = jnp.full((_L,), k, jnp.int32)
                vid = plsc.load_gather(ind_b, [rows, col])
                ib[pl.ds(g * _L, _L)] = jnp.clip(vid, 0, n_vtx - 1)

        for ib, ab in ((i0_b, a0_b), (i1_b, a1_b), (i2_b, a2_b)):
            for j in range(4):
                pltpu.async_copy(
                    attrs.at[ib.at[pl.ds(j * 128, 128)]],
                    ab.at[pl.ds(j * 128, 128)], sem_attr)

    def back(line, scal, ibuf, obuf, do_drain_out):
        drain_attr()

        @pl.when(do_drain_out)
        def _():
            drain_out(ibuf, obuf)

        @plsc.parallel_loop(0, _G, unroll=3)
        def blend(g):
            rows = g * _L + iota
            tw = (g // 8) * 1024
            wib = (g % 8) * _L
            u16 = scal[0, pl.ds(g * _L, _L)]
            v16 = scal[1, pl.ds(g * _L, _L)]
            w16 = scal[2, pl.ds(g * _L, _L)]
            d0 = scal[3, pl.ds(g * _L, _L)]
            d1 = scal[4, pl.ds(g * _L, _L)]
            d2 = scal[5, pl.ds(g * _L, _L)]
            d3 = scal[6, pl.ds(g * _L, _L)]
            for k in range(_A):
                ck = jnp.full((_L,), k, jnp.int32)
                a0k = plsc.load_gather(a0_b, [rows, ck])
                a1k = plsc.load_gather(a1_b, [rows, ck])
                a2k = plsc.load_gather(a2_b, [rows, ck])
                imgk = a0k * u16 + a1k * v16 + a2k * w16
                ibuf[pl.ds((k // 8) * 4096 + tw + (k % 8) * 128 + wib, _L)] = imgk
                da = a0k - a2k
                db = a1k - a2k
                dadx = da * d0 + db * d2
                dady = da * d1 + db * d3
                ce, co = 2 * k, 2 * k + 1
                obuf[pl.ds((ce // 8) * 4096 + tw + (ce % 8) * 128 + wib, _L)] = dadx
                obuf[pl.ds((co // 8) * 4096 + tw + (co % 8) * 128 + wib, _L)] = dady

        pltpu.async_copy(ibuf, out_img.at[pl.ds(line * _ISLAB, _ISLAB)], sem_out)
        pltpu.async_copy(obuf, out_der.at[pl.ds(line * _DSLAB, _DSLAB)], sem_out)

    issue_in(base, ra, dva)
    issue_in(base + 1, rb, dvb)

    def step(t, _):
        c0 = base + 2 * t
        c1 = c0 + 1
        ind0 = front(ra, dva, scal_a)

        @pl.when(t > 0)
        def _():
            back(c1 - 2, scal_b, img_c, dout_c, t > 1)

        mid(ind0)

        @pl.when(t + 1 < n_super)
        def _():
            issue_in(c0 + 2, ra, dva)

        ind1 = front(rb, dvb, scal_b)
        back(c0, scal_a, img_a, dout_a, t > 0)
        mid(ind1)

        @pl.when(t + 1 < n_super)
        def _():
            issue_in(c1 + 2, rb, dvb)

        return ()

    lax.fori_loop(0, n_super, step, ())

    back(base + n_lines - 1, scal_b, img_c, dout_c, n_super > 1)
    drain_out(img_a, dout_a)
    drain_out(img_c, dout_c)


@jax.jit
def _run(attributes, rast_lin, indices16, deriv_lin):
    n_pix = rast_lin.shape[0] // 4
    mesh = plsc.VectorSubcoreMesh(
        core_axis_name="c", subcore_axis_name="s", num_cores=2, num_subcores=16
    )
    kern = functools.partial(
        pl.kernel,
        out_type=[
            jax.ShapeDtypeStruct((n_pix * _A,), jnp.float32),
            jax.ShapeDtypeStruct((n_pix * 2 * _A,), jnp.float32),
        ],
        mesh=mesh,
        compiler_params=pltpu.CompilerParams(
            needs_layout_passes=False,
            use_tc_tiling_on_sc=False,
        ),
        scratch_types=[
            pltpu.VMEM((_P * 4,), jnp.float32),
            pltpu.VMEM((_P * 4,), jnp.float32),
            pltpu.VMEM((_P * 4,), jnp.float32),
            pltpu.VMEM((_P * 4,), jnp.float32),
            pltpu.VMEM((_P,), jnp.int32),
            pltpu.VMEM((_P, 16), jnp.int32),
            pltpu.VMEM((_P,), jnp.int32),
            pltpu.VMEM((_P,), jnp.int32),
            pltpu.VMEM((_P,), jnp.int32),
            pltpu.VMEM((_P, _A), jnp.float32),
            pltpu.VMEM((_P, _A), jnp.float32),
            pltpu.VMEM((_P, _A), jnp.float32),
            pltpu.VMEM((7, _P), jnp.float32),
            pltpu.VMEM((7, _P), jnp.float32),
            pltpu.VMEM((_ISLAB,), jnp.float32),
            pltpu.VMEM((_ISLAB,), jnp.float32),
            pltpu.VMEM((_DSLAB,), jnp.float32),
            pltpu.VMEM((_DSLAB,), jnp.float32),
            pltpu.SemaphoreType.DMA,
            pltpu.SemaphoreType.DMA,
            pltpu.SemaphoreType.DMA,
            pltpu.SemaphoreType.DMA,
        ],
    )(_body)
    return kern(rast_lin, deriv_lin, attributes, indices16)


def kernel(attributes, rasterized, indices, derivatives):
    b, h, w = rasterized.shape[:3]
    rast_lin = rasterized.reshape(b, h, w // 128, 128, 4)
    rast_lin = rast_lin.transpose(0, 1, 2, 4, 3).reshape(-1)
    deriv_lin = derivatives.reshape(b, h, w // 128, 128, 4)
    deriv_lin = deriv_lin.transpose(0, 1, 2, 4, 3).reshape(-1)
    inds16 = jnp.pad(indices, ((0, 0), (0, 16 - indices.shape[1])))
    img, der = _run(attributes, rast_lin, inds16, deriv_lin)
    img = img.reshape(b, h, 2, w // 128, 8, 128)
    img = img.transpose(0, 1, 3, 5, 2, 4).reshape(b, h, w, _A)
    der = der.reshape(b, h, 4, w // 128, 8, 128)
    der = der.transpose(0, 1, 3, 5, 2, 4).reshape(b, h, w, 2 * _A)
    return {"image": img, "derivatives": der}


# speedup vs baseline: 1.0489x; 1.0489x over previous
import functools
import jax
import jax.numpy as jnp
from jax import lax
from jax.experimental import pallas as pl
from jax.experimental.pallas import tpu as pltpu
from jax.experimental.pallas import tpu_sc as plsc

_A = 16
_L = 16
_NW = 32
_P = 512
_G = _P // _L
_ISLAB = _P * _A
_DSLAB = _P * 2 * _A


def _body(rast, deriv, attrs, inds, out_img, out_der,
          ra, rb, dva, dvb,
          tri_b, ind_b, i0_b, i1_b, i2_b,
          a0_b, a1_b, a2_b,
          scal_a, scal_b,
          img_a, img_c, dout_a, dout_c,
          sem_in, sem_ind, sem_attr, sem_out):
    n_pix = rast.shape[0] // 4
    n_tri = inds.shape[0]
    n_vtx = attrs.shape[0]
    wid = lax.axis_index("s") * 2 + lax.axis_index("c")
    n_lines = (n_pix // _NW) // _P
    n_super = n_lines // 2
    base = wid * n_lines
    iota = lax.iota(jnp.int32, _L)

    def issue_in(line, rbuf, dbuf):
        pltpu.async_copy(rast.at[pl.ds(line * 4 * _P, 4 * _P)], rbuf, sem_in)
        pltpu.async_copy(deriv.at[pl.ds(line * 4 * _P, 4 * _P)], dbuf, sem_in)

    def drain_in(rbuf, dbuf):
        pltpu.make_async_copy(rast.at[pl.ds(0, 4 * _P)], rbuf, sem_in).wait()
        pltpu.make_async_copy(deriv.at[pl.ds(0, 4 * _P)], dbuf, sem_in).wait()

    def drain_attr():
        for ab in (a0_b, a1_b, a2_b):
            pltpu.make_async_copy(attrs.at[pl.ds(0, _P)], ab, sem_attr).wait()

    def drain_out(ibuf, obuf):
        pltpu.make_async_copy(ibuf, out_img.at[pl.ds(0, _ISLAB)], sem_out).wait()
        pltpu.make_async_copy(obuf, out_der.at[pl.ds(0, _DSLAB)], sem_out).wait()

    def front(rbuf, dbuf, scal):
        drain_in(rbuf, dbuf)

        @plsc.parallel_loop(0, _G, unroll=4)
        def fields(g):
            off = (g // 8) * 512 + (g % 8) * _L
            trif = rbuf[pl.ds(off + 3 * 128, _L)]
            tri_i = trif.astype(jnp.int32)
            validf = jnp.where(tri_i > 0, 1.0, 0.0).astype(jnp.float32)
            tric = jnp.clip(tri_i - 1, 0, n_tri - 1)
            tri_b[pl.ds(g * _L, _L)] = tric
            u = rbuf[pl.ds(off, _L)]
            v = rbuf[pl.ds(off + 128, _L)]
            w = 1.0 - u - v
            scal[0, pl.ds(g * _L, _L)] = u * validf
            scal[1, pl.ds(g * _L, _L)] = v * validf
            scal[2, pl.ds(g * _L, _L)] = w * validf
            for k in range(4):
                dk = dbuf[pl.ds(off + k * 128, _L)]
                scal[3 + k, pl.ds(g * _L, _L)] = dk * validf

        return [pltpu.async_copy(
            inds.at[tri_b.at[pl.ds(j * 128, 128)]],
            ind_b.at[pl.ds(j * 128, 128)], sem_ind) for j in range(4)]

    def mid(ind_cps):
        for cp in ind_cps:
            cp.wait()

        @plsc.parallel_loop(0, _G, unroll=4)
        def split(g):
            rows = g * _L + iota
            for k, ib in ((0, i0_b), (1, i1_b), (2, i2_b)):
                col = jnp.full((_L,), k, jnp.int32)
                vid = plsc.load_gather(ind_b, [rows, col])
                ib[pl.ds(g * _L, _L)] = jnp.clip(vid, 0, n_vtx - 1)

        for ib, ab in ((i0_b, a0_b), (i1_b, a1_b), (i2_b, a2_b)):
            for j in range(4):
                pltpu.async_copy(
                    attrs.at[ib.at[pl.ds(j * 128, 128)]],
                    ab.at[pl.ds(j * 128, 128)], sem_attr)

    def back(line, scal, ibuf, obuf, do_drain_out):
        drain_attr()

        @pl.when(do_drain_out)
        def _():
            drain_out(ibuf, obuf)

        @plsc.parallel_loop(0, _G, unroll=2)
        def blend(g):
            rows = g * _L + iota
            tw = (g // 8) * 1024
            wib = (g % 8) * _L
            u16 = scal[0, pl.ds(g * _L, _L)]
            v16 = scal[1, pl.ds(g * _L, _L)]
            w16 = scal[2, pl.ds(g * _L, _L)]
            d0 = scal[3, pl.ds(g * _L, _L)]
            d1 = scal[4, pl.ds(g * _L, _L)]
            d2 = scal[5, pl.ds(g * _L, _L)]
            d3 = scal[6, pl.ds(g * _L, _L)]
            for k in range(_A):
                ck = jnp.full((_L,), k, jnp.int32)
                a0k = plsc.load_gather(a0_b, [rows, ck])
                a1k = plsc.load_gather(a1_b, [rows, ck])
                a2k = plsc.load_gather(a2_b, [rows, ck])
                imgk = a0k * u16 + a1k * v16 + a2k * w16
                ibuf[pl.ds((k // 8) * 4096 + tw + (k % 8) * 128 + wib, _L)] = imgk
                da = a0k - a2k
                db = a1k - a2k
                dadx = da * d0 + db * d2
                dady = da * d1 + db * d3
                ce, co = 2 * k, 2 * k + 1
                obuf[pl.ds((ce // 8) * 4096 + tw + (ce % 8) * 128 + wib, _L)] = dadx
                obuf[pl.ds((co // 8) * 4096 + tw + (co % 8) * 128 + wib, _L)] = dady

        pltpu.async_copy(ibuf, out_img.at[pl.ds(line * _ISLAB, _ISLAB)], sem_out)
        pltpu.async_copy(obuf, out_der.at[pl.ds(line * _DSLAB, _DSLAB)], sem_out)

    issue_in(base, ra, dva)
    issue_in(base + 1, rb, dvb)

    def step(t, _):
        c0 = base + 2 * t
        c1 = c0 + 1
        ind0 = front(ra, dva, scal_a)

        @pl.when(t > 0)
        def _():
            back(c1 - 2, scal_b, img_c, dout_c, t > 1)

        mid(ind0)

        @pl.when(t + 1 < n_super)
        def _():
            issue_in(c0 + 2, ra, dva)

        ind1 = front(rb, dvb, scal_b)
        back(c0, scal_a, img_a, dout_a, t > 0)
        mid(ind1)

        @pl.when(t + 1 < n_super)
        def _():
            issue_in(c1 + 2, rb, dvb)

        return ()

    lax.fori_loop(0, n_super, step, ())

    back(base + n_lines - 1, scal_b, img_c, dout_c, n_super > 1)
    drain_out(img_a, dout_a)
    drain_out(img_c, dout_c)


@jax.jit
def _run(attributes, rast_lin, indices16, deriv_lin):
    n_pix = rast_lin.shape[0] // 4
    mesh = plsc.VectorSubcoreMesh(
        core_axis_name="c", subcore_axis_name="s", num_cores=2, num_subcores=16
    )
    kern = functools.partial(
        pl.kernel,
        out_type=[
            jax.ShapeDtypeStruct((n_pix * _A,), jnp.float32),
            jax.ShapeDtypeStruct((n_pix * 2 * _A,), jnp.float32),
        ],
        mesh=mesh,
        compiler_params=pltpu.CompilerParams(
            needs_layout_passes=False,
            use_tc_tiling_on_sc=False,
        ),
        scratch_types=[
            pltpu.VMEM((_P * 4,), jnp.float32),
            pltpu.VMEM((_P * 4,), jnp.float32),
            pltpu.VMEM((_P * 4,), jnp.float32),
            pltpu.VMEM((_P * 4,), jnp.float32),
            pltpu.VMEM((_P,), jnp.int32),
            pltpu.VMEM((_P, 16), jnp.int32),
            pltpu.VMEM((_P,), jnp.int32),
            pltpu.VMEM((_P,), jnp.int32),
            pltpu.VMEM((_P,), jnp.int32),
            pltpu.VMEM((_P, _A), jnp.float32),
            pltpu.VMEM((_P, _A), jnp.float32),
            pltpu.VMEM((_P, _A), jnp.float32),
            pltpu.VMEM((7, _P), jnp.float32),
            pltpu.VMEM((7, _P), jnp.float32),
            pltpu.VMEM((_ISLAB,), jnp.float32),
            pltpu.VMEM((_ISLAB,), jnp.float32),
            pltpu.VMEM((_DSLAB,), jnp.float32),
            pltpu.VMEM((_DSLAB,), jnp.float32),
            pltpu.SemaphoreType.DMA,
            pltpu.SemaphoreType.DMA,
            pltpu.SemaphoreType.DMA,
            pltpu.SemaphoreType.DMA,
        ],
    )(_body)
    return kern(rast_lin, deriv_lin, attributes, indices16)


def kernel(attributes, rasterized, indices, derivatives):
    b, h, w = rasterized.shape[:3]
    rast_lin = rasterized.reshape(b, h, w // 128, 128, 4)
    rast_lin = rast_lin.transpose(0, 1, 2, 4, 3).reshape(-1)
    deriv_lin = derivatives.reshape(b, h, w // 128, 128, 4)
    deriv_lin = deriv_lin.transpose(0, 1, 2, 4, 3).reshape(-1)
    inds16 = jnp.pad(indices, ((0, 0), (0, 16 - indices.shape[1])))
    img, der = _run(attributes, rast_lin, inds16, deriv_lin)
    img = img.reshape(b, h, 2, w // 128, 8, 128)
    img = img.transpose(0, 1, 3, 5, 2, 4).reshape(b, h, w, _A)
    der = der.reshape(b, h, 4, w // 128, 8, 128)
    der = der.transpose(0, 1, 3, 5, 2, 4).reshape(b, h, w, 2 * _A)
    return {"image": img, "derivatives": der}
